# baseline (speedup 1.0000x reference)
import jax
import jax.numpy as jnp
from jax.experimental import pallas as pl

_BLOCK_ROWS = 2000


def _att_kernel(x_ref, w0_ref, w1_ref, o_ref):
    o_ref[...] = jnp.maximum(x_ref[...] * w0_ref[...], 0.0) * w1_ref[...]


def kernel(features, w0, w1):
    n, d = features.shape
    return pl.pallas_call(
        _att_kernel,
        grid=(n // _BLOCK_ROWS,),
        in_specs=[
            pl.BlockSpec((_BLOCK_ROWS, d), lambda i: (i, 0)),
            pl.BlockSpec((1, d), lambda i: (0, 0)),
            pl.BlockSpec((1, d), lambda i: (0, 0)),
        ],
        out_specs=pl.BlockSpec((_BLOCK_ROWS, d), lambda i: (i, 0)),
        out_shape=jax.ShapeDtypeStruct((n, d), features.dtype),
    )(features, w0.reshape(1, d), w1.reshape(1, d))


# speedup vs baseline: 1.5740x; 1.5740x over previous
import jax
import jax.numpy as jnp
from jax.experimental import pallas as pl

_BLOCK_ROWS = 10000


def _att_kernel(x_ref, w0_ref, w1_ref, o_ref):
    o_ref[...] = jnp.maximum(x_ref[...] * w0_ref[...], 0.0) * w1_ref[...]


def kernel(features, w0, w1):
    n, d = features.shape
    return pl.pallas_call(
        _att_kernel,
        grid=(n // _BLOCK_ROWS,),
        in_specs=[
            pl.BlockSpec((_BLOCK_ROWS, d), lambda i: (i, 0)),
            pl.BlockSpec((1, d), lambda i: (0, 0)),
            pl.BlockSpec((1, d), lambda i: (0, 0)),
        ],
        out_specs=pl.BlockSpec((_BLOCK_ROWS, d), lambda i: (i, 0)),
        out_shape=jax.ShapeDtypeStruct((n, d), features.dtype),
    )(features, w0.reshape(1, d), w1.reshape(1, d))


# speedup vs baseline: 1.6471x; 1.0465x over previous
import jax
import jax.numpy as jnp
from jax.experimental import pallas as pl

_BLOCK_ROWS = 20000


def _att_kernel(x_ref, w0_ref, w1_ref, o_ref):
    o_ref[...] = jnp.maximum(x_ref[...] * w0_ref[...], 0.0) * w1_ref[...]


def kernel(features, w0, w1):
    n, d = features.shape
    return pl.pallas_call(
        _att_kernel,
        grid=(n // _BLOCK_ROWS,),
        in_specs=[
            pl.BlockSpec((_BLOCK_ROWS, d), lambda i: (i, 0)),
            pl.BlockSpec((1, d), lambda i: (0, 0)),
            pl.BlockSpec((1, d), lambda i: (0, 0)),
        ],
        out_specs=pl.BlockSpec((_BLOCK_ROWS, d), lambda i: (i, 0)),
        out_shape=jax.ShapeDtypeStruct((n, d), features.dtype),
    )(features, w0.reshape(1, d), w1.reshape(1, d))


# speedup vs baseline: 1.6573x; 1.0062x over previous
import jax
import jax.numpy as jnp
from jax.experimental import pallas as pl

_BLOCK_ROWS = 25000


def _att_kernel(x_ref, w0_ref, w1_ref, o_ref):
    o_ref[...] = jnp.maximum(x_ref[...] * w0_ref[...], 0.0) * w1_ref[...]


def kernel(features, w0, w1):
    n, d = features.shape
    return pl.pallas_call(
        _att_kernel,
        grid=(n // _BLOCK_ROWS,),
        in_specs=[
            pl.BlockSpec((_BLOCK_ROWS, d), lambda i: (i, 0)),
            pl.BlockSpec((1, d), lambda i: (0, 0)),
            pl.BlockSpec((1, d), lambda i: (0, 0)),
        ],
        out_specs=pl.BlockSpec((_BLOCK_ROWS, d), lambda i: (i, 0)),
        out_shape=jax.ShapeDtypeStruct((n, d), features.dtype),
    )(features, w0.reshape(1, d), w1.reshape(1, d))
